# baseline (speedup 1.0000x reference)
import functools

import jax
import jax.numpy as jnp
from jax import lax
from jax.experimental import pallas as pl
from jax.experimental.pallas import tpu as pltpu
from jax.experimental.pallas import tpu_sc as plsc

N = 10000
E = 160000
NE = 10000
IN = 256
OUT = 256
HEADS = 8
DH = OUT // HEADS

NC = 2
NS = 16
HALF = 128

EPT = E // NS
KB = 80
NFB = EPT // KB

RB = 24
RPT = 624
RTAIL = NE - RPT * NS

_f32 = jnp.float32
_i32 = jnp.int32

_mesh = plsc.VectorSubcoreMesh(core_axis_name="c", subcore_axis_name="s")

_GDN = lax.GatherDimensionNumbers(
    offset_dims=(), collapsed_slice_dims=(0,), start_index_map=(0,))


def _splat(r, h):
    return lax.gather(r, jnp.full((16, 1), h, _i32), _GDN, (1,),
                      mode=lax.GatherScatterMode.PROMISE_IN_BOUNDS)


def _ring_pipeline(issue_idx, wait_idx, issue_gathers, wait_gathers, process,
                   wait_scatters):
    issue_idx(0, 0)
    issue_idx(1, 1)

    def stage(b, k, static):
        def wsc():
            wait_scatters(b - 2, (k + 2) % 4, k % 2)

        def drain():
            wait_gathers(b - 1, (k + 3) % 4, (k + 1) % 2)
            process(b - 1, (k + 3) % 4, (k + 1) % 2)

        def prefetch():
            issue_idx(b + 2, (k + 2) % 4)

        if static:
            if b >= 2:
                wsc()
            wait_idx(b, k % 4)
            issue_gathers(b, k % 4, k % 2)
            if b >= 1:
                drain()
            if b + 2 < NFB:
                prefetch()
        else:
            pl.when(b >= 2)(wsc)
            wait_idx(b, k % 4)
            issue_gathers(b, k % 4, k % 2)
            pl.when(b >= 1)(drain)
            pl.when(b + 2 < NFB)(prefetch)

    def g_body(g, _):
        for k in range(4):
            stage(4 * g + k, k, False)
        return 0
    lax.fori_loop(0, NFB // 4, g_body, 0)

    for b in range(4 * (NFB // 4), NFB):
        stage(b, b % 4, True)

    last = NFB - 1
    wait_gathers(last, last % 4, last % 2)
    process(last, last % 4, last % 2)
    wait_scatters(last - 1, (last - 1) % 4, (last - 1) % 2)
    wait_scatters(last, last % 4, last % 2)


def _zero_rows(nbuf_v, dbuf_v):
    zero16 = jnp.zeros((16,), _f32)

    def _zrow(t, _):
        nbuf_v[t // 8, pl.ds((t % 8) * 16, 16)] = zero16
        return 0
    lax.fori_loop(0, RB * 8, _zrow, 0)

    def _zdeg(i, _):
        dbuf_v[i, :] = zero16
        return 0
    lax.fori_loop(0, RB, _zdeg, 0)


def _zero_shared(sid, nbuf_v, dbuf_v, big_sh, small_sh):
    for blk in range(RPT // RB):
        r0 = sid * RPT + blk * RB
        pltpu.sync_copy(nbuf_v, big_sh.at[pl.ds(r0, RB)])
        pltpu.sync_copy(dbuf_v, small_sh.at[pl.ds(r0, RB)])

    @pl.when(sid == NS - 1)
    def _():
        pltpu.sync_copy(nbuf_v.at[pl.ds(0, RTAIL)],
                        big_sh.at[pl.ds(RPT * NS, RTAIL)])
        pltpu.sync_copy(dbuf_v.at[pl.ds(0, RTAIL)],
                        small_sh.at[pl.ds(RPT * NS, RTAIL)])


_ILV = None


@functools.partial(
    pl.kernel,
    out_type=[
        jax.ShapeDtypeStruct((NE, HALF), jnp.bfloat16),
        jax.ShapeDtypeStruct((NE, HALF), jnp.bfloat16),
        jax.ShapeDtypeStruct((NE, 16), _f32),
    ],
    mesh=_mesh,
    scratch_types=[
        [pltpu.VMEM((KB,), _i32)] * 4,
        [pltpu.VMEM((KB,), _i32)] * 4,
        [pltpu.VMEM((KB, HALF), jnp.bfloat16)] * 2,
        pltpu.VMEM((KB, HALF), _f32),
        [pltpu.VMEM((KB, 16), _f32)] * 2,
        pltpu.VMEM((RB, HALF), _f32),
        pltpu.VMEM((RB, 16), _f32),
        pltpu.VMEM((RB, HALF), jnp.bfloat16),
        pltpu.VMEM_SHARED((NE, HALF), _f32),
        pltpu.VMEM_SHARED((NE, 16), _f32),
        [pltpu.SemaphoreType.DMA] * 4,
        [pltpu.SemaphoreType.DMA] * 2,
        [pltpu.SemaphoreType.DMA] * 2,
    ],
    compiler_params=pltpu.CompilerParams(use_tc_tiling_on_sc=False,
                                         needs_layout_passes=False),
)
def _sc_edge_mean(nidx_hbm, eidx_hbm, xh0_hbm, xh1_hbm, s2x_hbm,
                  ef0_hbm, ef1_hbm, se_hbm,
                  nidx_s, eidx_s, rows_s, stage_v, s2g_s,
                  nbuf_v, dbuf_v, ebuf_v, ef_sh, se_sh, isem, gsem, ssem):
    cid = lax.axis_index("c")
    sid = lax.axis_index("s")

    _zero_rows(nbuf_v, dbuf_v)
    _zero_shared(sid, nbuf_v, dbuf_v, ef_sh, se_sh)
    plsc.subcore_barrier()

    base = sid * EPT
    ilv = plsc.PackFormat.INTERLEAVED

    def issue_idx(b, s):
        off = base + b * KB
        pltpu.async_copy(nidx_hbm.at[pl.ds(off, KB)], nidx_s[s], isem[s])
        pltpu.async_copy(eidx_hbm.at[pl.ds(off, KB)], eidx_s[s], isem[s])

    def wait_idx(b, s):
        off = base + b * KB
        pltpu.make_async_copy(nidx_hbm.at[pl.ds(off, KB)], nidx_s[s], isem[s]).wait()
        pltpu.make_async_copy(eidx_hbm.at[pl.ds(off, KB)], eidx_s[s], isem[s]).wait()

    def issue_gathers(b, i4, g2):
        pltpu.async_copy(s2x_hbm.at[nidx_s[i4]], s2g_s[g2], gsem[g2])

        @pl.when(cid == 0)
        def _():
            pltpu.async_copy(xh0_hbm.at[nidx_s[i4]], rows_s[g2], gsem[g2])

        @pl.when(cid == 1)
        def _():
            pltpu.async_copy(xh1_hbm.at[nidx_s[i4]], rows_s[g2], gsem[g2])

    def wait_gathers(b, i4, g2):
        pltpu.make_async_copy(s2x_hbm.at[nidx_s[i4]], s2g_s[g2], gsem[g2]).wait()

        @pl.when(cid == 0)
        def _():
            pltpu.make_async_copy(xh0_hbm.at[nidx_s[i4]], rows_s[g2], gsem[g2]).wait()

        @pl.when(cid == 1)
        def _():
            pltpu.make_async_copy(xh1_hbm.at[nidx_s[i4]], rows_s[g2], gsem[g2]).wait()

    def process(b, i4, g2):
        rows = rows_s[g2]

        @plsc.parallel_loop(0, KB, unroll=4)
        def _(i):
            for g in range(4):
                u, v = plsc.unpack(rows[i, pl.ds(32 * g, 32)], format=ilv)
                stage_v[i, pl.ds(32 * g, 16)] = u
                stage_v[i, pl.ds(32 * g + 16, 16)] = v

        pltpu.async_copy(s2g_s[g2], se_sh.at[eidx_s[i4]], ssem[g2], add=True)
        pltpu.sync_copy(stage_v, ef_sh.at[eidx_s[i4]], add=True)

    def wait_scatters(b, i4, g2):
        pltpu.make_async_copy(s2g_s[g2], se_sh.at[eidx_s[i4]], ssem[g2]).wait()

    _ring_pipeline(issue_idx, wait_idx, issue_gathers, wait_gathers, process,
                   wait_scatters)

    plsc.subcore_barrier()

    def _norm_out(r0, nrows):
        pltpu.sync_copy(ef_sh.at[pl.ds(r0, nrows)], nbuf_v.at[pl.ds(0, nrows)])
        pltpu.sync_copy(se_sh.at[pl.ds(r0, nrows)], dbuf_v.at[pl.ds(0, nrows)])

        @plsc.parallel_loop(0, nrows, unroll=2)
        def _(i):
            d = 1.0 / jnp.maximum(_splat(dbuf_v[i, :], 8), 1.0)
            dbuf_v[i, :] = dbuf_v[i, :] * d
            for g in range(4):
                a = nbuf_v[i, pl.ds(32 * g, 16)] * d
                b2 = nbuf_v[i, pl.ds(32 * g + 16, 16)] * d
                ebuf_v[i, pl.ds(32 * g, 32)] = plsc.pack(
                    a, b2, format=ilv).astype(jnp.bfloat16)

        @pl.when(cid == 0)
        def _():
            pltpu.sync_copy(ebuf_v.at[pl.ds(0, nrows)], ef0_hbm.at[pl.ds(r0, nrows)])
            pltpu.sync_copy(dbuf_v.at[pl.ds(0, nrows)], se_hbm.at[pl.ds(r0, nrows)])

        @pl.when(cid == 1)
        def _():
            pltpu.sync_copy(ebuf_v.at[pl.ds(0, nrows)], ef1_hbm.at[pl.ds(r0, nrows)])

    for blk in range(RPT // RB):
        _norm_out(sid * RPT + blk * RB, RB)

    @pl.when(sid == NS - 1)
    def _():
        _norm_out(RPT * NS, RTAIL)


@functools.partial(
    pl.kernel,
    out_type=[
        jax.ShapeDtypeStruct((N, HALF), _f32),
        jax.ShapeDtypeStruct((N, HALF), _f32),
    ],
    mesh=_mesh,
    scratch_types=[
        [pltpu.VMEM((KB,), _i32)] * 4,
        [pltpu.VMEM((KB,), _i32)] * 4,
        [pltpu.VMEM((KB, 16), _f32)] * 2,
        [pltpu.VMEM((KB, 16), _f32)] * 2,
        [pltpu.VMEM((KB, HALF), jnp.bfloat16)] * 2,
        pltpu.VMEM((KB, HALF), _f32),
        [pltpu.VMEM((KB, 16), _f32)] * 2,
        pltpu.VMEM((RB, HALF), _f32),
        pltpu.VMEM((RB, 16), _f32),
        pltpu.VMEM_SHARED((N, HALF), _f32),
        pltpu.VMEM_SHARED((N, 16), _f32),
        [pltpu.SemaphoreType.DMA] * 4,
        [pltpu.SemaphoreType.DMA] * 2,
        [pltpu.SemaphoreType.DMA] * 2,
    ],
    compiler_params=pltpu.CompilerParams(use_tc_tiling_on_sc=False,
                                         needs_layout_passes=False),
)
def _sc_attn_agg(nidx_hbm, eidx_hbm, sx_hbm, se_hbm, ef0_hbm, ef1_hbm,
                 o0_hbm, o1_hbm,
                 nidx_s, eidx_s, sxg_s, seg_s, rows_s, stage_v, aexp_s,
                 nbuf_v, dbuf_v, out_sh, den_sh, isem, gsem, ssem):
    cid = lax.axis_index("c")
    sid = lax.axis_index("s")

    _zero_rows(nbuf_v, dbuf_v)
    _zero_shared(sid, nbuf_v, dbuf_v, out_sh, den_sh)
    plsc.subcore_barrier()

    base = sid * EPT

    def issue_idx(b, s):
        off = base + b * KB
        pltpu.async_copy(nidx_hbm.at[pl.ds(off, KB)], nidx_s[s], isem[s])
        pltpu.async_copy(eidx_hbm.at[pl.ds(off, KB)], eidx_s[s], isem[s])

    def wait_idx(b, s):
        off = base + b * KB
        pltpu.make_async_copy(nidx_hbm.at[pl.ds(off, KB)], nidx_s[s], isem[s]).wait()
        pltpu.make_async_copy(eidx_hbm.at[pl.ds(off, KB)], eidx_s[s], isem[s]).wait()

    def issue_gathers(b, i3, g2):
        pltpu.async_copy(sx_hbm.at[nidx_s[i3]], sxg_s[g2], gsem[g2])
        pltpu.async_copy(se_hbm.at[eidx_s[i3]], seg_s[g2], gsem[g2])

        @pl.when(cid == 0)
        def _():
            pltpu.async_copy(ef0_hbm.at[eidx_s[i3]], rows_s[g2], gsem[g2])

        @pl.when(cid == 1)
        def _():
            pltpu.async_copy(ef1_hbm.at[eidx_s[i3]], rows_s[g2], gsem[g2])

    def wait_gathers(b, i3, g2):
        pltpu.make_async_copy(sx_hbm.at[nidx_s[i3]], sxg_s[g2], gsem[g2]).wait()
        pltpu.make_async_copy(se_hbm.at[eidx_s[i3]], seg_s[g2], gsem[g2]).wait()

        @pl.when(cid == 0)
        def _():
            pltpu.make_async_copy(ef0_hbm.at[eidx_s[i3]], rows_s[g2], gsem[g2]).wait()

        @pl.when(cid == 1)
        def _():
            pltpu.make_async_copy(ef1_hbm.at[eidx_s[i3]], rows_s[g2], gsem[g2]).wait()

    ilv = plsc.PackFormat.INTERLEAVED

    def _weight_rows(sxg, seg, aexp, rows, stage):
        @plsc.parallel_loop(0, KB, unroll=4)
        def _(i):
            a = sxg[i, :] + seg[i, :]
            a = jnp.where(a > 0, a, 0.2 * a)
            ae = jnp.exp(a)
            aexp[i, :] = ae
            for g in range(4):
                w = _splat(ae, cid * 4 + g)
                u, v = plsc.unpack(rows[i, pl.ds(32 * g, 32)], format=ilv)
                stage[i, pl.ds(32 * g, 16)] = u * w
                stage[i, pl.ds(32 * g + 16, 16)] = v * w

    def process(b, i4, g2):
        _weight_rows(sxg_s[g2], seg_s[g2], aexp_s[g2], rows_s[g2], stage_v)
        pltpu.async_copy(aexp_s[g2], den_sh.at[nidx_s[i4]], ssem[g2], add=True)
        pltpu.sync_copy(stage_v, out_sh.at[nidx_s[i4]], add=True)

    def wait_scatters(b, i4, g2):
        pltpu.make_async_copy(aexp_s[g2], den_sh.at[nidx_s[i4]], ssem[g2]).wait()

    _ring_pipeline(issue_idx, wait_idx, issue_gathers, wait_gathers, process,
                   wait_scatters)

    plsc.subcore_barrier()

    def _final_out(r0, nrows):
        pltpu.sync_copy(out_sh.at[pl.ds(r0, nrows)], nbuf_v.at[pl.ds(0, nrows)])
        pltpu.sync_copy(den_sh.at[pl.ds(r0, nrows)], dbuf_v.at[pl.ds(0, nrows)])

        @plsc.parallel_loop(0, nrows, unroll=2)
        def _(i):
            r = 1.0 / jnp.maximum(dbuf_v[i, :], 1e-16)
            for hh in range(4):
                w = _splat(r, cid * 4 + hh)
                c0 = hh * 32
                nbuf_v[i, pl.ds(c0, 16)] = nbuf_v[i, pl.ds(c0, 16)] * w
                nbuf_v[i, pl.ds(c0 + 16, 16)] = nbuf_v[i, pl.ds(c0 + 16, 16)] * w

        @pl.when(cid == 0)
        def _():
            pltpu.sync_copy(nbuf_v.at[pl.ds(0, nrows)], o0_hbm.at[pl.ds(r0, nrows)])

        @pl.when(cid == 1)
        def _():
            pltpu.sync_copy(nbuf_v.at[pl.ds(0, nrows)], o1_hbm.at[pl.ds(r0, nrows)])

    for blk in range(RPT // RB):
        _final_out(sid * RPT + blk * RB, RB)

    @pl.when(sid == NS - 1)
    def _():
        _final_out(RPT * NS, RTAIL)


_BLK = 400


def _tc1_body(x_ref, w_ref, wp_ref, a1_ref, a2_ref,
              xb0_ref, xb1_ref, sx_ref, s2_ref):
    xh = jnp.dot(x_ref[:, :], w_ref[:, :], preferred_element_type=_f32)
    xhp = jnp.dot(x_ref[:, :], wp_ref[:, :], preferred_element_type=_f32)
    xb0_ref[:, :] = xhp[:, :HALF].astype(jnp.bfloat16)
    xb1_ref[:, :] = xhp[:, HALF:].astype(jnp.bfloat16)
    sx_ref[:, :] = jnp.dot(xh, a1_ref[:, :], preferred_element_type=_f32)
    pad = jnp.concatenate(
        [jnp.zeros((1, 8), _f32), jnp.ones((1, 8), _f32)], axis=1)
    s2_ref[:, :] = jnp.dot(xh, a2_ref[:, :], preferred_element_type=_f32) + pad


def _tc1(x, W, Wp, A1p, A2p):
    grid = (N // _BLK,)
    return pl.pallas_call(
        _tc1_body,
        grid=grid,
        in_specs=[
            pl.BlockSpec((_BLK, IN), lambda i: (i, 0)),
            pl.BlockSpec((IN, OUT), lambda i: (0, 0)),
            pl.BlockSpec((IN, OUT), lambda i: (0, 0)),
            pl.BlockSpec((OUT, 16), lambda i: (0, 0)),
            pl.BlockSpec((OUT, 16), lambda i: (0, 0)),
        ],
        out_specs=[
            pl.BlockSpec((_BLK, HALF), lambda i: (i, 0)),
            pl.BlockSpec((_BLK, HALF), lambda i: (i, 0)),
            pl.BlockSpec((_BLK, 16), lambda i: (i, 0)),
            pl.BlockSpec((_BLK, 16), lambda i: (i, 0)),
        ],
        out_shape=[
            jax.ShapeDtypeStruct((N, HALF), jnp.bfloat16),
            jax.ShapeDtypeStruct((N, HALF), jnp.bfloat16),
            jax.ShapeDtypeStruct((N, 16), _f32),
            jax.ShapeDtypeStruct((N, 16), _f32),
        ],
    )(x, W, Wp, A1p, A2p)


def _tc3_body(o0_ref, o1_ref, x_ref, b_ref, g_ref, be_ref, y_ref):
    conv = jnp.concatenate([o0_ref[:, :], o1_ref[:, :]], axis=1)
    out = conv + b_ref[:, :] + x_ref[:, :]
    mu = jnp.mean(out, axis=1, keepdims=True)
    c = out - mu
    var = jnp.mean(c * c, axis=1, keepdims=True)
    y = c * lax.rsqrt(var + 1e-5) * g_ref[:, :] + be_ref[:, :]
    y_ref[:, :] = jnp.where(y > 0, y, jnp.exp(jnp.minimum(y, 0.0)) - 1.0)


def _tc3(o0, o1, x, b2, g2, be2):
    grid = (N // _BLK,)
    return pl.pallas_call(
        _tc3_body,
        grid=grid,
        in_specs=[
            pl.BlockSpec((_BLK, HALF), lambda i: (i, 0)),
            pl.BlockSpec((_BLK, HALF), lambda i: (i, 0)),
            pl.BlockSpec((_BLK, OUT), lambda i: (i, 0)),
            pl.BlockSpec((1, OUT), lambda i: (0, 0)),
            pl.BlockSpec((1, OUT), lambda i: (0, 0)),
            pl.BlockSpec((1, OUT), lambda i: (0, 0)),
        ],
        out_specs=pl.BlockSpec((_BLK, OUT), lambda i: (i, 0)),
        out_shape=jax.ShapeDtypeStruct((N, OUT), _f32),
    )(o0, o1, x, b2, g2, be2)


def kernel(x, hyperedge_index, W, b, att, gamma, beta):
    node_idx = hyperedge_index[0].astype(_i32)
    edge_idx = hyperedge_index[1].astype(_i32)

    eye = jnp.eye(HEADS, dtype=_f32)
    A1 = (att[:, :DH, None] * eye[:, None, :]).reshape(OUT, HEADS)
    A2 = (att[:, DH:, None] * eye[:, None, :]).reshape(OUT, HEADS)
    A1p = jnp.pad(A1, ((0, 0), (0, 16 - HEADS)))
    A2p = jnp.pad(A2, ((0, 0), (0, 16 - HEADS)))

    perm = []
    for g in range(OUT // 32):
        for i in range(16):
            perm.extend([32 * g + i, 32 * g + 16 + i])
    Wp = W[:, jnp.array(perm, dtype=jnp.int32)]

    xb0, xb1, sx, s2x = _tc1(x, W, Wp, A1p, A2p)
    ef0, ef1, se = _sc_edge_mean(node_idx, edge_idx, xb0, xb1, s2x)
    o0, o1 = _sc_attn_agg(node_idx, edge_idx, sx, se, ef0, ef1)
    return _tc3(o0, o1, x, b.reshape(1, OUT), gamma.reshape(1, OUT),
                beta.reshape(1, OUT))


# speedup vs baseline: 1.0137x; 1.0137x over previous
import functools

import jax
import jax.numpy as jnp
from jax import lax
from jax.experimental import pallas as pl
from jax.experimental.pallas import tpu as pltpu
from jax.experimental.pallas import tpu_sc as plsc

N = 10000
E = 160000
NE = 10000
IN = 256
OUT = 256
HEADS = 8
DH = OUT // HEADS

NC = 2
NS = 16
HALF = 128

EPT = E // NS
KB = 80
NFB = EPT // KB

RB = 24
RPT = 624
RTAIL = NE - RPT * NS

_f32 = jnp.float32
_i32 = jnp.int32

_mesh = plsc.VectorSubcoreMesh(core_axis_name="c", subcore_axis_name="s")

_GDN = lax.GatherDimensionNumbers(
    offset_dims=(), collapsed_slice_dims=(0,), start_index_map=(0,))


def _splat(r, h):
    return lax.gather(r, jnp.full((16, 1), h, _i32), _GDN, (1,),
                      mode=lax.GatherScatterMode.PROMISE_IN_BOUNDS)


def _ring_pipeline(issue_idx, wait_idx, issue_gathers, wait_gathers, process,
                   wait_scatters):
    issue_idx(0, 0)
    issue_idx(1, 1)

    def stage(b, k, static):
        def wsc():
            wait_scatters(b - 2, (k + 2) % 4, k % 2)

        def drain():
            wait_gathers(b - 1, (k + 3) % 4, (k + 1) % 2)
            process(b - 1, (k + 3) % 4, (k + 1) % 2)

        def prefetch():
            issue_idx(b + 2, (k + 2) % 4)

        if static:
            if b >= 2:
                wsc()
            wait_idx(b, k % 4)
            issue_gathers(b, k % 4, k % 2)
            if b >= 1:
                drain()
            if b + 2 < NFB:
                prefetch()
        else:
            pl.when(b >= 2)(wsc)
            wait_idx(b, k % 4)
            issue_gathers(b, k % 4, k % 2)
            pl.when(b >= 1)(drain)
            pl.when(b + 2 < NFB)(prefetch)

    def g_body(g, _):
        for k in range(4):
            stage(4 * g + k, k, False)
        return 0
    lax.fori_loop(0, NFB // 4, g_body, 0)

    for b in range(4 * (NFB // 4), NFB):
        stage(b, b % 4, True)

    last = NFB - 1
    wait_gathers(last, last % 4, last % 2)
    process(last, last % 4, last % 2)
    wait_scatters(last - 1, (last - 1) % 4, (last - 1) % 2)
    wait_scatters(last, last % 4, last % 2)


def _zero_rows(nbuf_v, dbuf_v):
    zero16 = jnp.zeros((16,), _f32)

    def _zrow(t, _):
        nbuf_v[t // 8, pl.ds((t % 8) * 16, 16)] = zero16
        return 0
    lax.fori_loop(0, RB * 8, _zrow, 0)

    def _zdeg(i, _):
        dbuf_v[i, :] = zero16
        return 0
    lax.fori_loop(0, RB, _zdeg, 0)


def _zero_shared(sid, nbuf_v, dbuf_v, big_sh, small_sh):
    for blk in range(RPT // RB):
        r0 = sid * RPT + blk * RB
        pltpu.sync_copy(nbuf_v, big_sh.at[pl.ds(r0, RB)])
        pltpu.sync_copy(dbuf_v, small_sh.at[pl.ds(r0, RB)])

    @pl.when(sid == NS - 1)
    def _():
        pltpu.sync_copy(nbuf_v.at[pl.ds(0, RTAIL)],
                        big_sh.at[pl.ds(RPT * NS, RTAIL)])
        pltpu.sync_copy(dbuf_v.at[pl.ds(0, RTAIL)],
                        small_sh.at[pl.ds(RPT * NS, RTAIL)])


@functools.partial(
    pl.kernel,
    out_type=[
        jax.ShapeDtypeStruct((NE, HALF), _f32),
        jax.ShapeDtypeStruct((NE, HALF), _f32),
        jax.ShapeDtypeStruct((NE, 16), _f32),
    ],
    mesh=_mesh,
    scratch_types=[
        [pltpu.VMEM((KB,), _i32)] * 4,
        [pltpu.VMEM((KB,), _i32)] * 4,
        [pltpu.VMEM((KB, HALF), _f32)] * 2,
        [pltpu.VMEM((KB, 16), _f32)] * 2,
        pltpu.VMEM((RB, HALF), _f32),
        pltpu.VMEM((RB, 16), _f32),
        pltpu.VMEM_SHARED((NE, HALF), _f32),
        pltpu.VMEM_SHARED((NE, 16), _f32),
        [pltpu.SemaphoreType.DMA] * 4,
        [pltpu.SemaphoreType.DMA] * 2,
        [pltpu.SemaphoreType.DMA] * 2,
    ],
    compiler_params=pltpu.CompilerParams(use_tc_tiling_on_sc=False,
                                         needs_layout_passes=False),
)
def _sc_edge_mean(nidx_hbm, eidx_hbm, xh0_hbm, xh1_hbm, s2x_hbm,
                  ef0_hbm, ef1_hbm, se_hbm,
                  nidx_s, eidx_s, rows_s, s2g_s,
                  nbuf_v, dbuf_v, ef_sh, se_sh, isem, gsem, ssem):
    cid = lax.axis_index("c")
    sid = lax.axis_index("s")

    _zero_rows(nbuf_v, dbuf_v)
    _zero_shared(sid, nbuf_v, dbuf_v, ef_sh, se_sh)
    plsc.subcore_barrier()

    base = sid * EPT

    def issue_idx(b, s):
        off = base + b * KB
        pltpu.async_copy(nidx_hbm.at[pl.ds(off, KB)], nidx_s[s], isem[s])
        pltpu.async_copy(eidx_hbm.at[pl.ds(off, KB)], eidx_s[s], isem[s])

    def wait_idx(b, s):
        off = base + b * KB
        pltpu.make_async_copy(nidx_hbm.at[pl.ds(off, KB)], nidx_s[s], isem[s]).wait()
        pltpu.make_async_copy(eidx_hbm.at[pl.ds(off, KB)], eidx_s[s], isem[s]).wait()

    def issue_gathers(b, i4, g2):
        pltpu.async_copy(s2x_hbm.at[nidx_s[i4]], s2g_s[g2], gsem[g2])

        @pl.when(cid == 0)
        def _():
            pltpu.async_copy(xh0_hbm.at[nidx_s[i4]], rows_s[g2], gsem[g2])

        @pl.when(cid == 1)
        def _():
            pltpu.async_copy(xh1_hbm.at[nidx_s[i4]], rows_s[g2], gsem[g2])

    def wait_gathers(b, i4, g2):
        pltpu.make_async_copy(s2x_hbm.at[nidx_s[i4]], s2g_s[g2], gsem[g2]).wait()

        @pl.when(cid == 0)
        def _():
            pltpu.make_async_copy(xh0_hbm.at[nidx_s[i4]], rows_s[g2], gsem[g2]).wait()

        @pl.when(cid == 1)
        def _():
            pltpu.make_async_copy(xh1_hbm.at[nidx_s[i4]], rows_s[g2], gsem[g2]).wait()

    def process(b, i4, g2):
        pltpu.async_copy(rows_s[g2], ef_sh.at[eidx_s[i4]], ssem[g2], add=True)
        pltpu.async_copy(s2g_s[g2], se_sh.at[eidx_s[i4]], ssem[g2], add=True)

    def wait_scatters(b, i4, g2):
        pltpu.make_async_copy(rows_s[g2], ef_sh.at[eidx_s[i4]], ssem[g2]).wait()
        pltpu.make_async_copy(s2g_s[g2], se_sh.at[eidx_s[i4]], ssem[g2]).wait()

    _ring_pipeline(issue_idx, wait_idx, issue_gathers, wait_gathers, process,
                   wait_scatters)

    plsc.subcore_barrier()

    def _norm_out(r0, nrows):
        pltpu.sync_copy(ef_sh.at[pl.ds(r0, nrows)], nbuf_v.at[pl.ds(0, nrows)])
        pltpu.sync_copy(se_sh.at[pl.ds(r0, nrows)], dbuf_v.at[pl.ds(0, nrows)])

        @plsc.parallel_loop(0, nrows, unroll=2)
        def _(i):
            d = 1.0 / jnp.maximum(_splat(dbuf_v[i, :], 8), 1.0)
            dbuf_v[i, :] = dbuf_v[i, :] * d
            for v in range(8):
                nbuf_v[i, pl.ds(v * 16, 16)] = nbuf_v[i, pl.ds(v * 16, 16)] * d

        @pl.when(cid == 0)
        def _():
            pltpu.sync_copy(nbuf_v.at[pl.ds(0, nrows)], ef0_hbm.at[pl.ds(r0, nrows)])
            pltpu.sync_copy(dbuf_v.at[pl.ds(0, nrows)], se_hbm.at[pl.ds(r0, nrows)])

        @pl.when(cid == 1)
        def _():
            pltpu.sync_copy(nbuf_v.at[pl.ds(0, nrows)], ef1_hbm.at[pl.ds(r0, nrows)])

    for blk in range(RPT // RB):
        _norm_out(sid * RPT + blk * RB, RB)

    @pl.when(sid == NS - 1)
    def _():
        _norm_out(RPT * NS, RTAIL)


@functools.partial(
    pl.kernel,
    out_type=[
        jax.ShapeDtypeStruct((N, HALF), _f32),
        jax.ShapeDtypeStruct((N, HALF), _f32),
    ],
    mesh=_mesh,
    scratch_types=[
        [pltpu.VMEM((KB,), _i32)] * 4,
        [pltpu.VMEM((KB,), _i32)] * 4,
        [pltpu.VMEM((KB, 16), _f32)] * 2,
        [pltpu.VMEM((KB, 16), _f32)] * 2,
        [pltpu.VMEM((KB, HALF), _f32)] * 2,
        [pltpu.VMEM((KB, 16), _f32)] * 2,
        pltpu.VMEM((RB, HALF), _f32),
        pltpu.VMEM((RB, 16), _f32),
        pltpu.VMEM_SHARED((N, HALF), _f32),
        pltpu.VMEM_SHARED((N, 16), _f32),
        [pltpu.SemaphoreType.DMA] * 4,
        [pltpu.SemaphoreType.DMA] * 2,
        [pltpu.SemaphoreType.DMA] * 2,
    ],
    compiler_params=pltpu.CompilerParams(use_tc_tiling_on_sc=False,
                                         needs_layout_passes=False),
)
def _sc_attn_agg(nidx_hbm, eidx_hbm, sx_hbm, se_hbm, ef0_hbm, ef1_hbm,
                 o0_hbm, o1_hbm,
                 nidx_s, eidx_s, sxg_s, seg_s, rows_s, aexp_s,
                 nbuf_v, dbuf_v, out_sh, den_sh, isem, gsem, ssem):
    cid = lax.axis_index("c")
    sid = lax.axis_index("s")

    _zero_rows(nbuf_v, dbuf_v)
    _zero_shared(sid, nbuf_v, dbuf_v, out_sh, den_sh)
    plsc.subcore_barrier()

    base = sid * EPT

    def issue_idx(b, s):
        off = base + b * KB
        pltpu.async_copy(nidx_hbm.at[pl.ds(off, KB)], nidx_s[s], isem[s])
        pltpu.async_copy(eidx_hbm.at[pl.ds(off, KB)], eidx_s[s], isem[s])

    def wait_idx(b, s):
        off = base + b * KB
        pltpu.make_async_copy(nidx_hbm.at[pl.ds(off, KB)], nidx_s[s], isem[s]).wait()
        pltpu.make_async_copy(eidx_hbm.at[pl.ds(off, KB)], eidx_s[s], isem[s]).wait()

    def issue_gathers(b, i4, g2):
        pltpu.async_copy(sx_hbm.at[nidx_s[i4]], sxg_s[g2], gsem[g2])
        pltpu.async_copy(se_hbm.at[eidx_s[i4]], seg_s[g2], gsem[g2])

        @pl.when(cid == 0)
        def _():
            pltpu.async_copy(ef0_hbm.at[eidx_s[i4]], rows_s[g2], gsem[g2])

        @pl.when(cid == 1)
        def _():
            pltpu.async_copy(ef1_hbm.at[eidx_s[i4]], rows_s[g2], gsem[g2])

    def wait_gathers(b, i4, g2):
        pltpu.make_async_copy(sx_hbm.at[nidx_s[i4]], sxg_s[g2], gsem[g2]).wait()
        pltpu.make_async_copy(se_hbm.at[eidx_s[i4]], seg_s[g2], gsem[g2]).wait()

        @pl.when(cid == 0)
        def _():
            pltpu.make_async_copy(ef0_hbm.at[eidx_s[i4]], rows_s[g2], gsem[g2]).wait()

        @pl.when(cid == 1)
        def _():
            pltpu.make_async_copy(ef1_hbm.at[eidx_s[i4]], rows_s[g2], gsem[g2]).wait()

    def _weight_rows(sxg, seg, aexp, rows):
        @plsc.parallel_loop(0, KB, unroll=4)
        def _(i):
            a = sxg[i, :] + seg[i, :]
            a = jnp.where(a > 0, a, 0.2 * a)
            ae = jnp.exp(a)
            aexp[i, :] = ae
            for hh in range(4):
                w = _splat(ae, cid * 4 + hh)
                c0 = hh * 32
                rows[i, pl.ds(c0, 16)] = rows[i, pl.ds(c0, 16)] * w
                rows[i, pl.ds(c0 + 16, 16)] = rows[i, pl.ds(c0 + 16, 16)] * w

    def process(b, i4, g2):
        _weight_rows(sxg_s[g2], seg_s[g2], aexp_s[g2], rows_s[g2])
        pltpu.async_copy(aexp_s[g2], den_sh.at[nidx_s[i4]], ssem[g2], add=True)
        pltpu.async_copy(rows_s[g2], out_sh.at[nidx_s[i4]], ssem[g2], add=True)

    def wait_scatters(b, i4, g2):
        pltpu.make_async_copy(aexp_s[g2], den_sh.at[nidx_s[i4]], ssem[g2]).wait()
        pltpu.make_async_copy(rows_s[g2], out_sh.at[nidx_s[i4]], ssem[g2]).wait()

    _ring_pipeline(issue_idx, wait_idx, issue_gathers, wait_gathers, process,
                   wait_scatters)

    plsc.subcore_barrier()

    def _final_out(r0, nrows):
        pltpu.sync_copy(out_sh.at[pl.ds(r0, nrows)], nbuf_v.at[pl.ds(0, nrows)])
        pltpu.sync_copy(den_sh.at[pl.ds(r0, nrows)], dbuf_v.at[pl.ds(0, nrows)])

        @plsc.parallel_loop(0, nrows, unroll=2)
        def _(i):
            r = 1.0 / jnp.maximum(dbuf_v[i, :], 1e-16)
            for hh in range(4):
                w = _splat(r, cid * 4 + hh)
                c0 = hh * 32
                nbuf_v[i, pl.ds(c0, 16)] = nbuf_v[i, pl.ds(c0, 16)] * w
                nbuf_v[i, pl.ds(c0 + 16, 16)] = nbuf_v[i, pl.ds(c0 + 16, 16)] * w

        @pl.when(cid == 0)
        def _():
            pltpu.sync_copy(nbuf_v.at[pl.ds(0, nrows)], o0_hbm.at[pl.ds(r0, nrows)])

        @pl.when(cid == 1)
        def _():
            pltpu.sync_copy(nbuf_v.at[pl.ds(0, nrows)], o1_hbm.at[pl.ds(r0, nrows)])

    for blk in range(RPT // RB):
        _final_out(sid * RPT + blk * RB, RB)

    @pl.when(sid == NS - 1)
    def _():
        _final_out(RPT * NS, RTAIL)


_BLK = 400


def _tc1_body(x_ref, w_ref, a1_ref, a2_ref, xh0_ref, xh1_ref, sx_ref, s2_ref):
    xh = jnp.dot(x_ref[:, :], w_ref[:, :], preferred_element_type=_f32)
    xh0_ref[:, :] = xh[:, :HALF]
    xh1_ref[:, :] = xh[:, HALF:]
    sx_ref[:, :] = jnp.dot(xh, a1_ref[:, :], preferred_element_type=_f32)
    pad = jnp.concatenate(
        [jnp.zeros((1, 8), _f32), jnp.ones((1, 8), _f32)], axis=1)
    s2_ref[:, :] = jnp.dot(xh, a2_ref[:, :], preferred_element_type=_f32) + pad


def _tc1(x, W, A1p, A2p):
    grid = (N // _BLK,)
    return pl.pallas_call(
        _tc1_body,
        grid=grid,
        in_specs=[
            pl.BlockSpec((_BLK, IN), lambda i: (i, 0)),
            pl.BlockSpec((IN, OUT), lambda i: (0, 0)),
            pl.BlockSpec((OUT, 16), lambda i: (0, 0)),
            pl.BlockSpec((OUT, 16), lambda i: (0, 0)),
        ],
        out_specs=[
            pl.BlockSpec((_BLK, HALF), lambda i: (i, 0)),
            pl.BlockSpec((_BLK, HALF), lambda i: (i, 0)),
            pl.BlockSpec((_BLK, 16), lambda i: (i, 0)),
            pl.BlockSpec((_BLK, 16), lambda i: (i, 0)),
        ],
        out_shape=[
            jax.ShapeDtypeStruct((N, HALF), _f32),
            jax.ShapeDtypeStruct((N, HALF), _f32),
            jax.ShapeDtypeStruct((N, 16), _f32),
            jax.ShapeDtypeStruct((N, 16), _f32),
        ],
    )(x, W, A1p, A2p)


def _tc3_body(o0_ref, o1_ref, x_ref, b_ref, g_ref, be_ref, y_ref):
    conv = jnp.concatenate([o0_ref[:, :], o1_ref[:, :]], axis=1)
    out = conv + b_ref[:, :] + x_ref[:, :]
    mu = jnp.mean(out, axis=1, keepdims=True)
    c = out - mu
    var = jnp.mean(c * c, axis=1, keepdims=True)
    y = c * lax.rsqrt(var + 1e-5) * g_ref[:, :] + be_ref[:, :]
    y_ref[:, :] = jnp.where(y > 0, y, jnp.exp(jnp.minimum(y, 0.0)) - 1.0)


def _tc3(o0, o1, x, b2, g2, be2):
    grid = (N // _BLK,)
    return pl.pallas_call(
        _tc3_body,
        grid=grid,
        in_specs=[
            pl.BlockSpec((_BLK, HALF), lambda i: (i, 0)),
            pl.BlockSpec((_BLK, HALF), lambda i: (i, 0)),
            pl.BlockSpec((_BLK, OUT), lambda i: (i, 0)),
            pl.BlockSpec((1, OUT), lambda i: (0, 0)),
            pl.BlockSpec((1, OUT), lambda i: (0, 0)),
            pl.BlockSpec((1, OUT), lambda i: (0, 0)),
        ],
        out_specs=pl.BlockSpec((_BLK, OUT), lambda i: (i, 0)),
        out_shape=jax.ShapeDtypeStruct((N, OUT), _f32),
    )(o0, o1, x, b2, g2, be2)


def kernel(x, hyperedge_index, W, b, att, gamma, beta):
    node_idx = hyperedge_index[0].astype(_i32)
    edge_idx = hyperedge_index[1].astype(_i32)

    eye = jnp.eye(HEADS, dtype=_f32)
    A1 = (att[:, :DH, None] * eye[:, None, :]).reshape(OUT, HEADS)
    A2 = (att[:, DH:, None] * eye[:, None, :]).reshape(OUT, HEADS)
    A1p = jnp.pad(A1, ((0, 0), (0, 16 - HEADS)))
    A2p = jnp.pad(A2, ((0, 0), (0, 16 - HEADS)))

    xh0, xh1, sx, s2x = _tc1(x, W, A1p, A2p)
    ef0, ef1, se = _sc_edge_mean(node_idx, edge_idx, xh0, xh1, s2x)
    o0, o1 = _sc_attn_agg(node_idx, edge_idx, sx, se, ef0, ef1)
    return _tc3(o0, o1, x, b.reshape(1, OUT), gamma.reshape(1, OUT),
                beta.reshape(1, OUT))


# speedup vs baseline: 1.0419x; 1.0278x over previous
import functools

import jax
import jax.numpy as jnp
from jax import lax
from jax.experimental import pallas as pl
from jax.experimental.pallas import tpu as pltpu
from jax.experimental.pallas import tpu_sc as plsc

N = 10000
E = 160000
NE = 10000
IN = 256
OUT = 256
HEADS = 8
DH = OUT // HEADS

NC = 2
NS = 16
HALF = 128

EPT = E // NS
KB = 80
NFB = EPT // KB

RB = 48
RPT = 624
RTAIL = NE - RPT * NS

_f32 = jnp.float32
_i32 = jnp.int32

_mesh = plsc.VectorSubcoreMesh(core_axis_name="c", subcore_axis_name="s")

_GDN = lax.GatherDimensionNumbers(
    offset_dims=(), collapsed_slice_dims=(0,), start_index_map=(0,))


def _splat(r, h):
    return lax.gather(r, jnp.full((16, 1), h, _i32), _GDN, (1,),
                      mode=lax.GatherScatterMode.PROMISE_IN_BOUNDS)


def _ring_pipeline(issue_idx, wait_idx, issue_gathers, wait_gathers, process,
                   wait_scatters):
    issue_idx(0, 0)
    issue_idx(1, 1)

    def stage(b, k, static):
        def wsc():
            wait_scatters(b - 2, (k + 2) % 4, k % 2)

        def drain():
            wait_gathers(b - 1, (k + 3) % 4, (k + 1) % 2)
            process(b - 1, (k + 3) % 4, (k + 1) % 2)

        def prefetch():
            issue_idx(b + 2, (k + 2) % 4)

        if static:
            if b >= 2:
                wsc()
            wait_idx(b, k % 4)
            issue_gathers(b, k % 4, k % 2)
            if b >= 1:
                drain()
            if b + 2 < NFB:
                prefetch()
        else:
            pl.when(b >= 2)(wsc)
            wait_idx(b, k % 4)
            issue_gathers(b, k % 4, k % 2)
            pl.when(b >= 1)(drain)
            pl.when(b + 2 < NFB)(prefetch)

    def g_body(g, _):
        for k in range(4):
            stage(4 * g + k, k, False)
        return 0
    lax.fori_loop(0, NFB // 4, g_body, 0)

    for b in range(4 * (NFB // 4), NFB):
        stage(b, b % 4, True)

    last = NFB - 1
    wait_gathers(last, last % 4, last % 2)
    process(last, last % 4, last % 2)
    wait_scatters(last - 1, (last - 1) % 4, (last - 1) % 2)
    wait_scatters(last, last % 4, last % 2)


def _zero_rows(nbuf_v, dbuf_v):
    zero16 = jnp.zeros((16,), _f32)

    def _zrow(t, _):
        nbuf_v[t // 8, pl.ds((t % 8) * 16, 16)] = zero16
        return 0
    lax.fori_loop(0, RB * 8, _zrow, 0)

    def _zdeg(i, _):
        dbuf_v[i, :] = zero16
        return 0
    lax.fori_loop(0, RB, _zdeg, 0)


def _zero_shared(sid, nbuf_v, dbuf_v, big_sh, small_sh):
    for blk in range(RPT // RB):
        r0 = sid * RPT + blk * RB
        pltpu.sync_copy(nbuf_v, big_sh.at[pl.ds(r0, RB)])
        pltpu.sync_copy(dbuf_v, small_sh.at[pl.ds(r0, RB)])

    @pl.when(sid == NS - 1)
    def _():
        pltpu.sync_copy(nbuf_v.at[pl.ds(0, RTAIL)],
                        big_sh.at[pl.ds(RPT * NS, RTAIL)])
        pltpu.sync_copy(dbuf_v.at[pl.ds(0, RTAIL)],
                        small_sh.at[pl.ds(RPT * NS, RTAIL)])


@functools.partial(
    pl.kernel,
    out_type=[
        jax.ShapeDtypeStruct((NE, HALF), _f32),
        jax.ShapeDtypeStruct((NE, HALF), _f32),
    ],
    mesh=_mesh,
    scratch_types=[
        [pltpu.VMEM((KB,), _i32)] * 4,
        [pltpu.VMEM((KB,), _i32)] * 4,
        [pltpu.VMEM((KB, HALF), _f32)] * 2,
        pltpu.VMEM((KB, 16), _f32),
        pltpu.VMEM((RB, HALF), _f32),
        pltpu.VMEM((RB, 16), _f32),
        pltpu.VMEM_SHARED((NE, HALF), _f32),
        pltpu.VMEM_SHARED((NE, 16), _f32),
        [pltpu.SemaphoreType.DMA] * 4,
        [pltpu.SemaphoreType.DMA] * 2,
        [pltpu.SemaphoreType.DMA] * 2,
    ],
    compiler_params=pltpu.CompilerParams(use_tc_tiling_on_sc=False),
)
def _sc_edge_mean(nidx_hbm, eidx_hbm, xh0_hbm, xh1_hbm, ef0_hbm, ef1_hbm,
                  nidx_s, eidx_s, rows_s, ones_v,
                  nbuf_v, dbuf_v, ef_sh, deg_sh, isem, gsem, ssem):
    cid = lax.axis_index("c")
    sid = lax.axis_index("s")

    _zero_rows(nbuf_v, dbuf_v)

    one16 = jnp.ones((16,), _f32)

    def _ones(i, _):
        ones_v[i, :] = one16
        return 0
    lax.fori_loop(0, KB, _ones, 0)

    _zero_shared(sid, nbuf_v, dbuf_v, ef_sh, deg_sh)
    plsc.subcore_barrier()

    base = sid * EPT

    def issue_idx(b, s):
        off = base + b * KB
        pltpu.async_copy(nidx_hbm.at[pl.ds(off, KB)], nidx_s[s], isem[s])
        pltpu.async_copy(eidx_hbm.at[pl.ds(off, KB)], eidx_s[s], isem[s])

    def wait_idx(b, s):
        off = base + b * KB
        pltpu.make_async_copy(nidx_hbm.at[pl.ds(off, KB)], nidx_s[s], isem[s]).wait()
        pltpu.make_async_copy(eidx_hbm.at[pl.ds(off, KB)], eidx_s[s], isem[s]).wait()

    def issue_gathers(b, i3, g2):
        @pl.when(cid == 0)
        def _():
            pltpu.async_copy(xh0_hbm.at[nidx_s[i3]], rows_s[g2], gsem[g2])

        @pl.when(cid == 1)
        def _():
            pltpu.async_copy(xh1_hbm.at[nidx_s[i3]], rows_s[g2], gsem[g2])

    def wait_gathers(b, i3, g2):
        @pl.when(cid == 0)
        def _():
            pltpu.make_async_copy(xh0_hbm.at[nidx_s[i3]], rows_s[g2], gsem[g2]).wait()

        @pl.when(cid == 1)
        def _():
            pltpu.make_async_copy(xh1_hbm.at[nidx_s[i3]], rows_s[g2], gsem[g2]).wait()

    def process(b, i4, g2):
        pltpu.async_copy(rows_s[g2], ef_sh.at[eidx_s[i4]], ssem[g2], add=True)
        pltpu.async_copy(ones_v, deg_sh.at[eidx_s[i4]], ssem[g2], add=True)

    def wait_scatters(b, i4, g2):
        pltpu.make_async_copy(rows_s[g2], ef_sh.at[eidx_s[i4]], ssem[g2]).wait()
        pltpu.make_async_copy(ones_v, deg_sh.at[eidx_s[i4]], ssem[g2]).wait()

    _ring_pipeline(issue_idx, wait_idx, issue_gathers, wait_gathers, process,
                   wait_scatters)

    plsc.subcore_barrier()

    def _norm_out(r0, nrows):
        pltpu.sync_copy(ef_sh.at[pl.ds(r0, nrows)], nbuf_v.at[pl.ds(0, nrows)])
        pltpu.sync_copy(deg_sh.at[pl.ds(r0, nrows)], dbuf_v.at[pl.ds(0, nrows)])

        @plsc.parallel_loop(0, nrows, unroll=2)
        def _(i):
            d = 1.0 / jnp.maximum(dbuf_v[i, :], 1.0)
            for v in range(8):
                nbuf_v[i, pl.ds(v * 16, 16)] = nbuf_v[i, pl.ds(v * 16, 16)] * d

        @pl.when(cid == 0)
        def _():
            pltpu.sync_copy(nbuf_v.at[pl.ds(0, nrows)], ef0_hbm.at[pl.ds(r0, nrows)])

        @pl.when(cid == 1)
        def _():
            pltpu.sync_copy(nbuf_v.at[pl.ds(0, nrows)], ef1_hbm.at[pl.ds(r0, nrows)])

    for blk in range(RPT // RB):
        _norm_out(sid * RPT + blk * RB, RB)

    @pl.when(sid == NS - 1)
    def _():
        _norm_out(RPT * NS, RTAIL)


@functools.partial(
    pl.kernel,
    out_type=[
        jax.ShapeDtypeStruct((N, HALF), _f32),
        jax.ShapeDtypeStruct((N, HALF), _f32),
    ],
    mesh=_mesh,
    scratch_types=[
        [pltpu.VMEM((KB,), _i32)] * 4,
        [pltpu.VMEM((KB,), _i32)] * 4,
        [pltpu.VMEM((KB, 16), _f32)] * 2,
        [pltpu.VMEM((KB, 16), _f32)] * 2,
        [pltpu.VMEM((KB, HALF), _f32)] * 2,
        [pltpu.VMEM((KB, 16), _f32)] * 2,
        pltpu.VMEM((RB, HALF), _f32),
        pltpu.VMEM((RB, 16), _f32),
        pltpu.VMEM_SHARED((N, HALF), _f32),
        pltpu.VMEM_SHARED((N, 16), _f32),
        [pltpu.SemaphoreType.DMA] * 4,
        [pltpu.SemaphoreType.DMA] * 2,
        [pltpu.SemaphoreType.DMA] * 2,
    ],
    compiler_params=pltpu.CompilerParams(use_tc_tiling_on_sc=False),
)
def _sc_attn_agg(nidx_hbm, eidx_hbm, sx_hbm, se_hbm, ef0_hbm, ef1_hbm,
                 o0_hbm, o1_hbm,
                 nidx_s, eidx_s, sxg_s, seg_s, rows_s, aexp_s,
                 nbuf_v, dbuf_v, out_sh, den_sh, isem, gsem, ssem):
    cid = lax.axis_index("c")
    sid = lax.axis_index("s")

    _zero_rows(nbuf_v, dbuf_v)
    _zero_shared(sid, nbuf_v, dbuf_v, out_sh, den_sh)
    plsc.subcore_barrier()

    base = sid * EPT

    def issue_idx(b, s):
        off = base + b * KB
        pltpu.async_copy(nidx_hbm.at[pl.ds(off, KB)], nidx_s[s], isem[s])
        pltpu.async_copy(eidx_hbm.at[pl.ds(off, KB)], eidx_s[s], isem[s])

    def wait_idx(b, s):
        off = base + b * KB
        pltpu.make_async_copy(nidx_hbm.at[pl.ds(off, KB)], nidx_s[s], isem[s]).wait()
        pltpu.make_async_copy(eidx_hbm.at[pl.ds(off, KB)], eidx_s[s], isem[s]).wait()

    def issue_gathers(b, i3, g2):
        pltpu.async_copy(sx_hbm.at[nidx_s[i3]], sxg_s[g2], gsem[g2])
        pltpu.async_copy(se_hbm.at[eidx_s[i3]], seg_s[g2], gsem[g2])

        @pl.when(cid == 0)
        def _():
            pltpu.async_copy(ef0_hbm.at[eidx_s[i3]], rows_s[g2], gsem[g2])

        @pl.when(cid == 1)
        def _():
            pltpu.async_copy(ef1_hbm.at[eidx_s[i3]], rows_s[g2], gsem[g2])

    def wait_gathers(b, i3, g2):
        pltpu.make_async_copy(sx_hbm.at[nidx_s[i3]], sxg_s[g2], gsem[g2]).wait()
        pltpu.make_async_copy(se_hbm.at[eidx_s[i3]], seg_s[g2], gsem[g2]).wait()

        @pl.when(cid == 0)
        def _():
            pltpu.make_async_copy(ef0_hbm.at[eidx_s[i3]], rows_s[g2], gsem[g2]).wait()

        @pl.when(cid == 1)
        def _():
            pltpu.make_async_copy(ef1_hbm.at[eidx_s[i3]], rows_s[g2], gsem[g2]).wait()

    def _weight_rows(sxg, seg, aexp, rows):
        @plsc.parallel_loop(0, KB, unroll=8)
        def _(i):
            a = sxg[i, :] + seg[i, :]
            a = jnp.where(a > 0, a, 0.2 * a)
            ae = jnp.exp(a)
            aexp[i, :] = ae
            for hh in range(4):
                w = _splat(ae, cid * 4 + hh)
                c0 = hh * 32
                rows[i, pl.ds(c0, 16)] = rows[i, pl.ds(c0, 16)] * w
                rows[i, pl.ds(c0 + 16, 16)] = rows[i, pl.ds(c0 + 16, 16)] * w

    def process(b, i4, g2):
        _weight_rows(sxg_s[g2], seg_s[g2], aexp_s[g2], rows_s[g2])
        pltpu.async_copy(aexp_s[g2], den_sh.at[nidx_s[i4]], ssem[g2], add=True)
        pltpu.async_copy(rows_s[g2], out_sh.at[nidx_s[i4]], ssem[g2], add=True)

    def wait_scatters(b, i4, g2):
        pltpu.make_async_copy(aexp_s[g2], den_sh.at[nidx_s[i4]], ssem[g2]).wait()
        pltpu.make_async_copy(rows_s[g2], out_sh.at[nidx_s[i4]], ssem[g2]).wait()

    _ring_pipeline(issue_idx, wait_idx, issue_gathers, wait_gathers, process,
                   wait_scatters)

    plsc.subcore_barrier()

    def _final_out(r0, nrows):
        pltpu.sync_copy(out_sh.at[pl.ds(r0, nrows)], nbuf_v.at[pl.ds(0, nrows)])
        pltpu.sync_copy(den_sh.at[pl.ds(r0, nrows)], dbuf_v.at[pl.ds(0, nrows)])

        @plsc.parallel_loop(0, nrows, unroll=2)
        def _(i):
            r = 1.0 / jnp.maximum(dbuf_v[i, :], 1e-16)
            for hh in range(4):
                w = _splat(r, cid * 4 + hh)
                c0 = hh * 32
                nbuf_v[i, pl.ds(c0, 16)] = nbuf_v[i, pl.ds(c0, 16)] * w
                nbuf_v[i, pl.ds(c0 + 16, 16)] = nbuf_v[i, pl.ds(c0 + 16, 16)] * w

        @pl.when(cid == 0)
        def _():
            pltpu.sync_copy(nbuf_v.at[pl.ds(0, nrows)], o0_hbm.at[pl.ds(r0, nrows)])

        @pl.when(cid == 1)
        def _():
            pltpu.sync_copy(nbuf_v.at[pl.ds(0, nrows)], o1_hbm.at[pl.ds(r0, nrows)])

    for blk in range(RPT // RB):
        _final_out(sid * RPT + blk * RB, RB)

    @pl.when(sid == NS - 1)
    def _():
        _final_out(RPT * NS, RTAIL)


_BLK = 400


def _tc1_body(x_ref, w_ref, a_ref, xh0_ref, xh1_ref, sx_ref):
    xh = jnp.dot(x_ref[:, :], w_ref[:, :], preferred_element_type=_f32)
    xh0_ref[:, :] = xh[:, :HALF]
    xh1_ref[:, :] = xh[:, HALF:]
    sx_ref[:, :] = jnp.dot(xh, a_ref[:, :], preferred_element_type=_f32)


def _tc1(x, W, A1p):
    grid = (N // _BLK,)
    return pl.pallas_call(
        _tc1_body,
        grid=grid,
        in_specs=[
            pl.BlockSpec((_BLK, IN), lambda i: (i, 0)),
            pl.BlockSpec((IN, OUT), lambda i: (0, 0)),
            pl.BlockSpec((OUT, 16), lambda i: (0, 0)),
        ],
        out_specs=[
            pl.BlockSpec((_BLK, HALF), lambda i: (i, 0)),
            pl.BlockSpec((_BLK, HALF), lambda i: (i, 0)),
            pl.BlockSpec((_BLK, 16), lambda i: (i, 0)),
        ],
        out_shape=[
            jax.ShapeDtypeStruct((N, HALF), _f32),
            jax.ShapeDtypeStruct((N, HALF), _f32),
            jax.ShapeDtypeStruct((N, 16), _f32),
        ],
    )(x, W, A1p)


def _tc2_body(ef0_ref, ef1_ref, a_ref, b_ref, se_ref):
    se_ref[:, :] = (
        jnp.dot(ef0_ref[:, :], a_ref[:, :], preferred_element_type=_f32)
        + jnp.dot(ef1_ref[:, :], b_ref[:, :], preferred_element_type=_f32)
    )


def _tc2(ef0, ef1, A2a, A2b):
    grid = (NE // _BLK,)
    return pl.pallas_call(
        _tc2_body,
        grid=grid,
        in_specs=[
            pl.BlockSpec((_BLK, HALF), lambda i: (i, 0)),
            pl.BlockSpec((_BLK, HALF), lambda i: (i, 0)),
            pl.BlockSpec((HALF, 16), lambda i: (0, 0)),
            pl.BlockSpec((HALF, 16), lambda i: (0, 0)),
        ],
        out_specs=pl.BlockSpec((_BLK, 16), lambda i: (i, 0)),
        out_shape=jax.ShapeDtypeStruct((NE, 16), _f32),
    )(ef0, ef1, A2a, A2b)


def _tc3_body(o0_ref, o1_ref, x_ref, b_ref, g_ref, be_ref, y_ref):
    conv = jnp.concatenate([o0_ref[:, :], o1_ref[:, :]], axis=1)
    out = conv + b_ref[:, :] + x_ref[:, :]
    mu = jnp.mean(out, axis=1, keepdims=True)
    c = out - mu
    var = jnp.mean(c * c, axis=1, keepdims=True)
    y = c * lax.rsqrt(var + 1e-5) * g_ref[:, :] + be_ref[:, :]
    y_ref[:, :] = jnp.where(y > 0, y, jnp.exp(jnp.minimum(y, 0.0)) - 1.0)


def _tc3(o0, o1, x, b2, g2, be2):
    grid = (N // _BLK,)
    return pl.pallas_call(
        _tc3_body,
        grid=grid,
        in_specs=[
            pl.BlockSpec((_BLK, HALF), lambda i: (i, 0)),
            pl.BlockSpec((_BLK, HALF), lambda i: (i, 0)),
            pl.BlockSpec((_BLK, OUT), lambda i: (i, 0)),
            pl.BlockSpec((1, OUT), lambda i: (0, 0)),
            pl.BlockSpec((1, OUT), lambda i: (0, 0)),
            pl.BlockSpec((1, OUT), lambda i: (0, 0)),
        ],
        out_specs=pl.BlockSpec((_BLK, OUT), lambda i: (i, 0)),
        out_shape=jax.ShapeDtypeStruct((N, OUT), _f32),
    )(o0, o1, x, b2, g2, be2)


def kernel(x, hyperedge_index, W, b, att, gamma, beta):
    node_idx = hyperedge_index[0].astype(_i32)
    edge_idx = hyperedge_index[1].astype(_i32)

    eye = jnp.eye(HEADS, dtype=_f32)
    A1 = (att[:, :DH, None] * eye[:, None, :]).reshape(OUT, HEADS)
    A2 = (att[:, DH:, None] * eye[:, None, :]).reshape(OUT, HEADS)
    A1p = jnp.pad(A1, ((0, 0), (0, 16 - HEADS)))
    A2p = jnp.pad(A2, ((0, 0), (0, 16 - HEADS)))

    xh0, xh1, sx = _tc1(x, W, A1p)
    ef0, ef1 = _sc_edge_mean(node_idx, edge_idx, xh0, xh1)
    se = _tc2(ef0, ef1, A2p[:HALF], A2p[HALF:])
    o0, o1 = _sc_attn_agg(node_idx, edge_idx, sx, se, ef0, ef1)
    return _tc3(o0, o1, x, b.reshape(1, OUT), gamma.reshape(1, OUT),
                beta.reshape(1, OUT))


# speedup vs baseline: 1.0487x; 1.0066x over previous
import functools

import jax
import jax.numpy as jnp
from jax import lax
from jax.experimental import pallas as pl
from jax.experimental.pallas import tpu as pltpu
from jax.experimental.pallas import tpu_sc as plsc

N = 10000
E = 160000
NE = 10000
IN = 256
OUT = 256
HEADS = 8
DH = OUT // HEADS

NC = 2
NS = 16
HALF = 128

EPT = E // NS
KB = 80
NFB = EPT // KB

RB = 48
RPT = 624
RTAIL = NE - RPT * NS

_f32 = jnp.float32
_i32 = jnp.int32

_mesh = plsc.VectorSubcoreMesh(core_axis_name="c", subcore_axis_name="s")

_GDN = lax.GatherDimensionNumbers(
    offset_dims=(), collapsed_slice_dims=(0,), start_index_map=(0,))


def _splat(r, h):
    return lax.gather(r, jnp.full((16, 1), h, _i32), _GDN, (1,),
                      mode=lax.GatherScatterMode.PROMISE_IN_BOUNDS)


def _ring_pipeline(issue_idx, wait_idx, issue_gathers, wait_gathers, process,
                   wait_scatters):
    issue_idx(0, 0)
    issue_idx(1, 1)

    def stage(b, k, static):
        def wsc():
            wait_scatters(b - 2, (k + 2) % 4, k % 2)

        def drain():
            wait_gathers(b - 1, (k + 3) % 4, (k + 1) % 2)
            process(b - 1, (k + 3) % 4, (k + 1) % 2)

        def prefetch():
            issue_idx(b + 2, (k + 2) % 4)

        if static:
            if b >= 2:
                wsc()
            wait_idx(b, k % 4)
            issue_gathers(b, k % 4, k % 2)
            if b >= 1:
                drain()
            if b + 2 < NFB:
                prefetch()
        else:
            pl.when(b >= 2)(wsc)
            wait_idx(b, k % 4)
            issue_gathers(b, k % 4, k % 2)
            pl.when(b >= 1)(drain)
            pl.when(b + 2 < NFB)(prefetch)

    def g_body(g, _):
        for k in range(4):
            stage(4 * g + k, k, False)
        return 0
    lax.fori_loop(0, NFB // 4, g_body, 0)

    for b in range(4 * (NFB // 4), NFB):
        stage(b, b % 4, True)

    last = NFB - 1
    wait_gathers(last, last % 4, last % 2)
    process(last, last % 4, last % 2)
    wait_scatters(last - 1, (last - 1) % 4, (last - 1) % 2)
    wait_scatters(last, last % 4, last % 2)


def _zero_rows(nbuf_v, dbuf_v):
    zero16 = jnp.zeros((16,), _f32)

    def _zrow(t, _):
        nbuf_v[t // 8, pl.ds((t % 8) * 16, 16)] = zero16
        return 0
    lax.fori_loop(0, RB * 8, _zrow, 0)

    def _zdeg(i, _):
        dbuf_v[i, :] = zero16
        return 0
    lax.fori_loop(0, RB, _zdeg, 0)


def _zero_shared(sid, nbuf_v, dbuf_v, big_sh, small_sh):
    for blk in range(RPT // RB):
        r0 = sid * RPT + blk * RB
        pltpu.sync_copy(nbuf_v, big_sh.at[pl.ds(r0, RB)])
        pltpu.sync_copy(dbuf_v, small_sh.at[pl.ds(r0, RB)])

    @pl.when(sid == NS - 1)
    def _():
        pltpu.sync_copy(nbuf_v.at[pl.ds(0, RTAIL)],
                        big_sh.at[pl.ds(RPT * NS, RTAIL)])
        pltpu.sync_copy(dbuf_v.at[pl.ds(0, RTAIL)],
                        small_sh.at[pl.ds(RPT * NS, RTAIL)])


@functools.partial(
    pl.kernel,
    out_type=[
        jax.ShapeDtypeStruct((NE, HALF), _f32),
        jax.ShapeDtypeStruct((NE, HALF), _f32),
    ],
    mesh=_mesh,
    scratch_types=[
        [pltpu.VMEM((KB,), _i32)] * 4,
        [pltpu.VMEM((KB,), _i32)] * 4,
        [pltpu.VMEM((KB, HALF), _f32)] * 2,
        pltpu.VMEM((KB, 16), _f32),
        pltpu.VMEM((RB, HALF), _f32),
        pltpu.VMEM((RB, 16), _f32),
        pltpu.VMEM_SHARED((NE, HALF), _f32),
        pltpu.VMEM_SHARED((NE, 16), _f32),
        [pltpu.SemaphoreType.DMA] * 4,
        [pltpu.SemaphoreType.DMA] * 2,
        [pltpu.SemaphoreType.DMA] * 2,
    ],
    compiler_params=pltpu.CompilerParams(use_tc_tiling_on_sc=False),
)
def _sc_edge_mean(nidx_hbm, eidx_hbm, xh0_hbm, xh1_hbm, ef0_hbm, ef1_hbm,
                  nidx_s, eidx_s, rows_s, ones_v,
                  nbuf_v, dbuf_v, ef_sh, deg_sh, isem, gsem, ssem):
    cid = lax.axis_index("c")
    sid = lax.axis_index("s")

    _zero_rows(nbuf_v, dbuf_v)

    one16 = jnp.ones((16,), _f32)

    def _ones(i, _):
        ones_v[i, :] = one16
        return 0
    lax.fori_loop(0, KB, _ones, 0)

    _zero_shared(sid, nbuf_v, dbuf_v, ef_sh, deg_sh)
    plsc.subcore_barrier()

    base = sid * EPT

    def issue_idx(b, s):
        off = base + b * KB
        pltpu.async_copy(nidx_hbm.at[pl.ds(off, KB)], nidx_s[s], isem[s])
        pltpu.async_copy(eidx_hbm.at[pl.ds(off, KB)], eidx_s[s], isem[s])

    def wait_idx(b, s):
        off = base + b * KB
        pltpu.make_async_copy(nidx_hbm.at[pl.ds(off, KB)], nidx_s[s], isem[s]).wait()
        pltpu.make_async_copy(eidx_hbm.at[pl.ds(off, KB)], eidx_s[s], isem[s]).wait()

    def issue_gathers(b, i3, g2):
        @pl.when(cid == 0)
        def _():
            pltpu.async_copy(xh0_hbm.at[nidx_s[i3]], rows_s[g2], gsem[g2])

        @pl.when(cid == 1)
        def _():
            pltpu.async_copy(xh1_hbm.at[nidx_s[i3]], rows_s[g2], gsem[g2])

    def wait_gathers(b, i3, g2):
        @pl.when(cid == 0)
        def _():
            pltpu.make_async_copy(xh0_hbm.at[nidx_s[i3]], rows_s[g2], gsem[g2]).wait()

        @pl.when(cid == 1)
        def _():
            pltpu.make_async_copy(xh1_hbm.at[nidx_s[i3]], rows_s[g2], gsem[g2]).wait()

    def process(b, i4, g2):
        pltpu.async_copy(rows_s[g2], ef_sh.at[eidx_s[i4]], ssem[g2], add=True)
        pltpu.async_copy(ones_v, deg_sh.at[eidx_s[i4]], ssem[g2], add=True)

    def wait_scatters(b, i4, g2):
        pltpu.make_async_copy(rows_s[g2], ef_sh.at[eidx_s[i4]], ssem[g2]).wait()
        pltpu.make_async_copy(ones_v, deg_sh.at[eidx_s[i4]], ssem[g2]).wait()

    _ring_pipeline(issue_idx, wait_idx, issue_gathers, wait_gathers, process,
                   wait_scatters)

    plsc.subcore_barrier()

    def _norm_out(r0, nrows):
        pltpu.sync_copy(ef_sh.at[pl.ds(r0, nrows)], nbuf_v.at[pl.ds(0, nrows)])
        pltpu.sync_copy(deg_sh.at[pl.ds(r0, nrows)], dbuf_v.at[pl.ds(0, nrows)])

        @plsc.parallel_loop(0, nrows, unroll=2)
        def _(i):
            d = 1.0 / jnp.maximum(dbuf_v[i, :], 1.0)
            for v in range(8):
                nbuf_v[i, pl.ds(v * 16, 16)] = nbuf_v[i, pl.ds(v * 16, 16)] * d

        @pl.when(cid == 0)
        def _():
            pltpu.sync_copy(nbuf_v.at[pl.ds(0, nrows)], ef0_hbm.at[pl.ds(r0, nrows)])

        @pl.when(cid == 1)
        def _():
            pltpu.sync_copy(nbuf_v.at[pl.ds(0, nrows)], ef1_hbm.at[pl.ds(r0, nrows)])

    for blk in range(RPT // RB):
        _norm_out(sid * RPT + blk * RB, RB)

    @pl.when(sid == NS - 1)
    def _():
        _norm_out(RPT * NS, RTAIL)


@functools.partial(
    pl.kernel,
    out_type=[
        jax.ShapeDtypeStruct((N, HALF), _f32),
        jax.ShapeDtypeStruct((N, HALF), _f32),
    ],
    mesh=_mesh,
    scratch_types=[
        [pltpu.VMEM((KB,), _i32)] * 4,
        [pltpu.VMEM((KB,), _i32)] * 4,
        [pltpu.VMEM((KB, 16), _f32)] * 2,
        [pltpu.VMEM((KB, 16), _f32)] * 2,
        [pltpu.VMEM((KB, HALF), _f32)] * 2,
        [pltpu.VMEM((KB, 16), _f32)] * 2,
        pltpu.VMEM((RB, HALF), _f32),
        pltpu.VMEM((RB, 16), _f32),
        pltpu.VMEM_SHARED((N, HALF), _f32),
        pltpu.VMEM_SHARED((N, 16), _f32),
        [pltpu.SemaphoreType.DMA] * 4,
        [pltpu.SemaphoreType.DMA] * 2,
        [pltpu.SemaphoreType.DMA] * 2,
    ],
    compiler_params=pltpu.CompilerParams(use_tc_tiling_on_sc=False),
)
def _sc_attn_agg(nidx_hbm, eidx_hbm, sx_hbm, se_hbm, ef0_hbm, ef1_hbm,
                 o0_hbm, o1_hbm,
                 nidx_s, eidx_s, sxg_s, seg_s, rows_s, aexp_s,
                 nbuf_v, dbuf_v, out_sh, den_sh, isem, gsem, ssem):
    cid = lax.axis_index("c")
    sid = lax.axis_index("s")

    _zero_rows(nbuf_v, dbuf_v)
    _zero_shared(sid, nbuf_v, dbuf_v, out_sh, den_sh)
    plsc.subcore_barrier()

    base = sid * EPT

    def issue_idx(b, s):
        off = base + b * KB
        pltpu.async_copy(nidx_hbm.at[pl.ds(off, KB)], nidx_s[s], isem[s])
        pltpu.async_copy(eidx_hbm.at[pl.ds(off, KB)], eidx_s[s], isem[s])

    def wait_idx(b, s):
        off = base + b * KB
        pltpu.make_async_copy(nidx_hbm.at[pl.ds(off, KB)], nidx_s[s], isem[s]).wait()
        pltpu.make_async_copy(eidx_hbm.at[pl.ds(off, KB)], eidx_s[s], isem[s]).wait()

    def issue_gathers(b, i3, g2):
        pltpu.async_copy(sx_hbm.at[nidx_s[i3]], sxg_s[g2], gsem[g2])
        pltpu.async_copy(se_hbm.at[eidx_s[i3]], seg_s[g2], gsem[g2])

        @pl.when(cid == 0)
        def _():
            pltpu.async_copy(ef0_hbm.at[eidx_s[i3]], rows_s[g2], gsem[g2])

        @pl.when(cid == 1)
        def _():
            pltpu.async_copy(ef1_hbm.at[eidx_s[i3]], rows_s[g2], gsem[g2])

    def wait_gathers(b, i3, g2):
        pltpu.make_async_copy(sx_hbm.at[nidx_s[i3]], sxg_s[g2], gsem[g2]).wait()
        pltpu.make_async_copy(se_hbm.at[eidx_s[i3]], seg_s[g2], gsem[g2]).wait()

        @pl.when(cid == 0)
        def _():
            pltpu.make_async_copy(ef0_hbm.at[eidx_s[i3]], rows_s[g2], gsem[g2]).wait()

        @pl.when(cid == 1)
        def _():
            pltpu.make_async_copy(ef1_hbm.at[eidx_s[i3]], rows_s[g2], gsem[g2]).wait()

    def _weight_rows(sxg, seg, aexp, rows):
        @plsc.parallel_loop(0, KB, unroll=4)
        def _(i):
            a = sxg[i, :] + seg[i, :]
            a = jnp.where(a > 0, a, 0.2 * a)
            ae = jnp.exp(a)
            aexp[i, :] = ae
            for hh in range(4):
                w = _splat(ae, cid * 4 + hh)
                c0 = hh * 32
                rows[i, pl.ds(c0, 16)] = rows[i, pl.ds(c0, 16)] * w
                rows[i, pl.ds(c0 + 16, 16)] = rows[i, pl.ds(c0 + 16, 16)] * w

    def process(b, i4, g2):
        _weight_rows(sxg_s[g2], seg_s[g2], aexp_s[g2], rows_s[g2])
        pltpu.async_copy(aexp_s[g2], den_sh.at[nidx_s[i4]], ssem[g2], add=True)
        pltpu.async_copy(rows_s[g2], out_sh.at[nidx_s[i4]], ssem[g2], add=True)

    def wait_scatters(b, i4, g2):
        pltpu.make_async_copy(aexp_s[g2], den_sh.at[nidx_s[i4]], ssem[g2]).wait()
        pltpu.make_async_copy(rows_s[g2], out_sh.at[nidx_s[i4]], ssem[g2]).wait()

    _ring_pipeline(issue_idx, wait_idx, issue_gathers, wait_gathers, process,
                   wait_scatters)

    plsc.subcore_barrier()

    def _final_out(r0, nrows):
        pltpu.sync_copy(out_sh.at[pl.ds(r0, nrows)], nbuf_v.at[pl.ds(0, nrows)])
        pltpu.sync_copy(den_sh.at[pl.ds(r0, nrows)], dbuf_v.at[pl.ds(0, nrows)])

        @plsc.parallel_loop(0, nrows, unroll=2)
        def _(i):
            r = 1.0 / jnp.maximum(dbuf_v[i, :], 1e-16)
            for hh in range(4):
                w = _splat(r, cid * 4 + hh)
                c0 = hh * 32
                nbuf_v[i, pl.ds(c0, 16)] = nbuf_v[i, pl.ds(c0, 16)] * w
                nbuf_v[i, pl.ds(c0 + 16, 16)] = nbuf_v[i, pl.ds(c0 + 16, 16)] * w

        @pl.when(cid == 0)
        def _():
            pltpu.sync_copy(nbuf_v.at[pl.ds(0, nrows)], o0_hbm.at[pl.ds(r0, nrows)])

        @pl.when(cid == 1)
        def _():
            pltpu.sync_copy(nbuf_v.at[pl.ds(0, nrows)], o1_hbm.at[pl.ds(r0, nrows)])

    for blk in range(RPT // RB):
        _final_out(sid * RPT + blk * RB, RB)

    @pl.when(sid == NS - 1)
    def _():
        _final_out(RPT * NS, RTAIL)


_BLK = 400


def _tc1_body(x_ref, w_ref, a_ref, xh0_ref, xh1_ref, sx_ref):
    xh = jnp.dot(x_ref[:, :], w_ref[:, :], preferred_element_type=_f32)
    xh0_ref[:, :] = xh[:, :HALF]
    xh1_ref[:, :] = xh[:, HALF:]
    sx_ref[:, :] = jnp.dot(xh, a_ref[:, :], preferred_element_type=_f32)


def _tc1(x, W, A1p):
    grid = (N // _BLK,)
    return pl.pallas_call(
        _tc1_body,
        grid=grid,
        in_specs=[
            pl.BlockSpec((_BLK, IN), lambda i: (i, 0)),
            pl.BlockSpec((IN, OUT), lambda i: (0, 0)),
            pl.BlockSpec((OUT, 16), lambda i: (0, 0)),
        ],
        out_specs=[
            pl.BlockSpec((_BLK, HALF), lambda i: (i, 0)),
            pl.BlockSpec((_BLK, HALF), lambda i: (i, 0)),
            pl.BlockSpec((_BLK, 16), lambda i: (i, 0)),
        ],
        out_shape=[
            jax.ShapeDtypeStruct((N, HALF), _f32),
            jax.ShapeDtypeStruct((N, HALF), _f32),
            jax.ShapeDtypeStruct((N, 16), _f32),
        ],
    )(x, W, A1p)


def _tc2_body(ef0_ref, ef1_ref, a_ref, b_ref, se_ref):
    se_ref[:, :] = (
        jnp.dot(ef0_ref[:, :], a_ref[:, :], preferred_element_type=_f32)
        + jnp.dot(ef1_ref[:, :], b_ref[:, :], preferred_element_type=_f32)
    )


def _tc2(ef0, ef1, A2a, A2b):
    grid = (NE // _BLK,)
    return pl.pallas_call(
        _tc2_body,
        grid=grid,
        in_specs=[
            pl.BlockSpec((_BLK, HALF), lambda i: (i, 0)),
            pl.BlockSpec((_BLK, HALF), lambda i: (i, 0)),
            pl.BlockSpec((HALF, 16), lambda i: (0, 0)),
            pl.BlockSpec((HALF, 16), lambda i: (0, 0)),
        ],
        out_specs=pl.BlockSpec((_BLK, 16), lambda i: (i, 0)),
        out_shape=jax.ShapeDtypeStruct((NE, 16), _f32),
    )(ef0, ef1, A2a, A2b)


def _tc3_body(o0_ref, o1_ref, x_ref, b_ref, g_ref, be_ref, y_ref):
    conv = jnp.concatenate([o0_ref[:, :], o1_ref[:, :]], axis=1)
    out = conv + b_ref[:, :] + x_ref[:, :]
    mu = jnp.mean(out, axis=1, keepdims=True)
    c = out - mu
    var = jnp.mean(c * c, axis=1, keepdims=True)
    y = c * lax.rsqrt(var + 1e-5) * g_ref[:, :] + be_ref[:, :]
    y_ref[:, :] = jnp.where(y > 0, y, jnp.exp(jnp.minimum(y, 0.0)) - 1.0)


def _tc3(o0, o1, x, b2, g2, be2):
    grid = (N // _BLK,)
    return pl.pallas_call(
        _tc3_body,
        grid=grid,
        in_specs=[
            pl.BlockSpec((_BLK, HALF), lambda i: (i, 0)),
            pl.BlockSpec((_BLK, HALF), lambda i: (i, 0)),
            pl.BlockSpec((_BLK, OUT), lambda i: (i, 0)),
            pl.BlockSpec((1, OUT), lambda i: (0, 0)),
            pl.BlockSpec((1, OUT), lambda i: (0, 0)),
            pl.BlockSpec((1, OUT), lambda i: (0, 0)),
        ],
        out_specs=pl.BlockSpec((_BLK, OUT), lambda i: (i, 0)),
        out_shape=jax.ShapeDtypeStruct((N, OUT), _f32),
    )(o0, o1, x, b2, g2, be2)


def kernel(x, hyperedge_index, W, b, att, gamma, beta):
    node_idx = hyperedge_index[0].astype(_i32)
    edge_idx = hyperedge_index[1].astype(_i32)

    eye = jnp.eye(HEADS, dtype=_f32)
    A1 = (att[:, :DH, None] * eye[:, None, :]).reshape(OUT, HEADS)
    A2 = (att[:, DH:, None] * eye[:, None, :]).reshape(OUT, HEADS)
    A1p = jnp.pad(A1, ((0, 0), (0, 16 - HEADS)))
    A2p = jnp.pad(A2, ((0, 0), (0, 16 - HEADS)))

    xh0, xh1, sx = _tc1(x, W, A1p)
    ef0, ef1 = _sc_edge_mean(node_idx, edge_idx, xh0, xh1)
    se = _tc2(ef0, ef1, A2p[:HALF], A2p[HALF:])
    o0, o1 = _sc_attn_agg(node_idx, edge_idx, sx, se, ef0, ef1)
    return _tc3(o0, o1, x, b.reshape(1, OUT), gamma.reshape(1, OUT),
                beta.reshape(1, OUT))
